# baseline (speedup 1.0000x reference)
import functools

import jax
import jax.numpy as jnp
from jax import lax
from jax.experimental import pallas as pl
from jax.experimental.pallas import tpu as pltpu
from jax.experimental.pallas import tpu_sc as plsc

_B = 1024
_L = 256
_LANES = 16
_NC = 2
_NS = 16
_NW = _NC * _NS
_B_SC = 384
_RPW = _B_SC // _NW
_LPAD = _L + 16

_sc_mesh = plsc.VectorSubcoreMesh(core_axis_name="c", subcore_axis_name="s")


@functools.partial(
    pl.kernel,
    mesh=_sc_mesh,
    compiler_params=pltpu.CompilerParams(
        use_tc_tiling_on_sc=False, needs_layout_passes=False
    ),
    out_type=jax.ShapeDtypeStruct((_NC, _LANES), jnp.float32),
    scratch_types=[
        pltpu.VMEM((_RPW, _LPAD), jnp.int32),
        pltpu.VMEM((_LANES,), jnp.float32),
        pltpu.VMEM((_NS, _LANES), jnp.float32),
        pltpu.VMEM_SHARED((_NS, _LANES), jnp.float32),
    ],
)
def _bpmll_sc(pk_hbm, out_hbm, pk_v, out_v, all_v, shared):
    cid = lax.axis_index("c")
    sid = lax.axis_index("s")
    wid = sid * _NC + cid
    row0 = wid * _RPW
    pltpu.sync_copy(
        pk_hbm.at[pl.ds(row0, _RPW)], pk_v.at[pl.ds(0, _RPW), pl.ds(0, _L)]
    )
    lanes = lax.iota(jnp.int32, _LANES)
    rows = jnp.minimum(lanes, _RPW - 1)

    def col_body(c, carry):
        s_neg, s_pos, cnt = carry
        cols = jnp.full((_LANES,), 0, jnp.int32) + c
        y = plsc.load_gather(pk_v, [rows, cols])
        t = y & 1
        x = plsc.bitcast(y & -2, jnp.float32)
        pos = t == 1
        s_neg = s_neg + jnp.where(pos, 0.0, jnp.exp(x))
        s_pos = s_pos + jnp.where(pos, jnp.exp(-x), 0.0)
        cnt = cnt + t
        return s_neg, s_pos, cnt

    zf = jnp.zeros((_LANES,), jnp.float32)
    zi = jnp.zeros((_LANES,), jnp.int32)
    s_neg, s_pos, cnt = lax.fori_loop(0, _L, col_body, (zf, zf, zi))
    npos = cnt.astype(jnp.float32)
    nneg = jnp.float32(_L) - npos
    partial = jnp.where(
        lanes < _RPW, s_neg * s_pos / (npos * nneg * jnp.float32(_B)), 0.0
    )
    out_v[:] = partial
    pltpu.sync_copy(out_v, shared.at[sid])
    plsc.subcore_barrier()

    @pl.when(sid == 0)
    def _():
        pltpu.sync_copy(shared, all_v)
        acc = jnp.zeros((_LANES,), jnp.float32)
        for i in range(_NS):
            acc = acc + all_v[i, :]
        total = jnp.sum(acc)
        out_v[:] = jnp.zeros((_LANES,), jnp.float32) + total
        pltpu.sync_copy(out_v, out_hbm.at[cid])


_TC_BLK = 128
_TC_STEPS = (_B - _B_SC) // _TC_BLK
_TC_OFF = _B_SC // _TC_BLK


def _tc_body(x_ref, t_ref, o_ref):
    i = pl.program_id(0)

    @pl.when(i == 0)
    def _():
        o_ref[...] = jnp.zeros((1, 1), jnp.float32)

    x = x_ref[...]
    pos = t_ref[...] == 1
    e = jnp.exp(x)
    en = jnp.exp(-x)
    s_neg = jnp.sum(jnp.where(pos, 0.0, e), axis=1)
    s_pos = jnp.sum(jnp.where(pos, en, 0.0), axis=1)
    npos = jnp.sum(pos.astype(jnp.float32), axis=1)
    loss = s_neg * s_pos / (npos * (jnp.float32(_L) - npos) * jnp.float32(_B))
    o_ref[...] += jnp.sum(loss)[None, None]


_tc_half = pl.pallas_call(
    _tc_body,
    grid=(_TC_STEPS,),
    in_specs=[
        pl.BlockSpec((_TC_BLK, _L), lambda i: (i + _TC_OFF, 0)),
        pl.BlockSpec((_TC_BLK, _L), lambda i: (i + _TC_OFF, 0)),
    ],
    out_specs=pl.BlockSpec((1, 1), lambda i: (0, 0)),
    out_shape=jax.ShapeDtypeStruct((1, 1), jnp.float32),
)


def kernel(input, target):
    tgt32 = target.astype(jnp.int32)
    packed = (
        jax.lax.bitcast_convert_type(input[:_B_SC], jnp.int32) & -2
    ) | tgt32[:_B_SC]
    sc_out = _bpmll_sc(packed)
    tc_out = _tc_half(input, tgt32)
    return sc_out[0, 0] + sc_out[1, 0] + tc_out[0, 0]


# speedup vs baseline: 1.0167x; 1.0167x over previous
import functools

import jax
import jax.numpy as jnp
from jax import lax
from jax.experimental import pallas as pl
from jax.experimental.pallas import tpu as pltpu
from jax.experimental.pallas import tpu_sc as plsc

_B = 1024
_L = 256
_LANES = 16
_NC = 2
_NS = 16
_NW = _NC * _NS
_B_SC = 384
_RPW = _B_SC // _NW
_LPAD = _L + 16

_sc_mesh = plsc.VectorSubcoreMesh(core_axis_name="c", subcore_axis_name="s")


@functools.partial(
    pl.kernel,
    mesh=_sc_mesh,
    compiler_params=pltpu.CompilerParams(
        use_tc_tiling_on_sc=False, needs_layout_passes=False
    ),
    out_type=jax.ShapeDtypeStruct((_NC, _LANES), jnp.float32),
    scratch_types=[
        pltpu.VMEM((_RPW, _LPAD), jnp.int32),
        pltpu.VMEM((_LANES,), jnp.float32),
        pltpu.VMEM((_NS, _LANES), jnp.float32),
        pltpu.VMEM_SHARED((_NS, _LANES), jnp.float32),
    ],
)
def _bpmll_sc(pk_hbm, out_hbm, pk_v, out_v, all_v, shared):
    cid = lax.axis_index("c")
    sid = lax.axis_index("s")
    wid = sid * _NC + cid
    row0 = wid * _RPW
    pltpu.sync_copy(
        pk_hbm.at[pl.ds(row0, _RPW)], pk_v.at[pl.ds(0, _RPW), pl.ds(0, _L)]
    )
    lanes = lax.iota(jnp.int32, _LANES)
    rows = jnp.minimum(lanes, _RPW - 1)

    def col_body(i, carry):
        sa_neg, sa_pos, cnta, sb_neg, sb_pos, cntb = carry
        c = i * 2
        colsa = jnp.full((_LANES,), 0, jnp.int32) + c
        ya = plsc.load_gather(pk_v, [rows, colsa])
        yb = plsc.load_gather(pk_v, [rows, colsa + 1])
        ta = ya & 1
        tb = yb & 1
        xa = plsc.bitcast(ya & -2, jnp.float32)
        xb = plsc.bitcast(yb & -2, jnp.float32)
        posa = ta == 1
        posb = tb == 1
        sa_neg = sa_neg + jnp.where(posa, 0.0, jnp.exp(xa))
        sb_neg = sb_neg + jnp.where(posb, 0.0, jnp.exp(xb))
        sa_pos = sa_pos + jnp.where(posa, jnp.exp(-xa), 0.0)
        sb_pos = sb_pos + jnp.where(posb, jnp.exp(-xb), 0.0)
        cnta = cnta + ta
        cntb = cntb + tb
        return sa_neg, sa_pos, cnta, sb_neg, sb_pos, cntb

    zf = jnp.zeros((_LANES,), jnp.float32)
    zi = jnp.zeros((_LANES,), jnp.int32)
    sa_neg, sa_pos, cnta, sb_neg, sb_pos, cntb = lax.fori_loop(
        0, _L // 2, col_body, (zf, zf, zi, zf, zf, zi)
    )
    s_neg = sa_neg + sb_neg
    s_pos = sa_pos + sb_pos
    cnt = cnta + cntb
    npos = cnt.astype(jnp.float32)
    nneg = jnp.float32(_L) - npos
    partial = jnp.where(
        lanes < _RPW, s_neg * s_pos / (npos * nneg * jnp.float32(_B)), 0.0
    )
    out_v[:] = partial
    pltpu.sync_copy(out_v, shared.at[sid])
    plsc.subcore_barrier()

    @pl.when(sid == 0)
    def _():
        pltpu.sync_copy(shared, all_v)
        acc = jnp.zeros((_LANES,), jnp.float32)
        for i in range(_NS):
            acc = acc + all_v[i, :]
        total = jnp.sum(acc)
        out_v[:] = jnp.zeros((_LANES,), jnp.float32) + total
        pltpu.sync_copy(out_v, out_hbm.at[cid])


_TC_BLK = 128
_TC_STEPS = (_B - _B_SC) // _TC_BLK
_TC_OFF = _B_SC // _TC_BLK


def _tc_body(x_ref, t_ref, o_ref):
    i = pl.program_id(0)

    @pl.when(i == 0)
    def _():
        o_ref[...] = jnp.zeros((1, 1), jnp.float32)

    x = x_ref[...]
    pos = t_ref[...] == 1
    e = jnp.exp(x)
    en = jnp.exp(-x)
    s_neg = jnp.sum(jnp.where(pos, 0.0, e), axis=1)
    s_pos = jnp.sum(jnp.where(pos, en, 0.0), axis=1)
    npos = jnp.sum(pos.astype(jnp.float32), axis=1)
    loss = s_neg * s_pos / (npos * (jnp.float32(_L) - npos) * jnp.float32(_B))
    o_ref[...] += jnp.sum(loss)[None, None]


_tc_half = pl.pallas_call(
    _tc_body,
    grid=(_TC_STEPS,),
    in_specs=[
        pl.BlockSpec((_TC_BLK, _L), lambda i: (i + _TC_OFF, 0)),
        pl.BlockSpec((_TC_BLK, _L), lambda i: (i + _TC_OFF, 0)),
    ],
    out_specs=pl.BlockSpec((1, 1), lambda i: (0, 0)),
    out_shape=jax.ShapeDtypeStruct((1, 1), jnp.float32),
)


def kernel(input, target):
    tgt32 = target.astype(jnp.int32)
    packed = (
        jax.lax.bitcast_convert_type(input[:_B_SC], jnp.int32) & -2
    ) | tgt32[:_B_SC]
    sc_out = _bpmll_sc(packed)
    tc_out = _tc_half(input, tgt32)
    return sc_out[0, 0] + sc_out[1, 0] + tc_out[0, 0]


# speedup vs baseline: 1.0179x; 1.0012x over previous
import functools

import jax
import jax.numpy as jnp
from jax import lax
from jax.experimental import pallas as pl
from jax.experimental.pallas import tpu as pltpu
from jax.experimental.pallas import tpu_sc as plsc

_B = 1024
_L = 256
_LANES = 16
_NC = 2
_NS = 16
_NW = _NC * _NS
_B_SC = 384
_RPW = _B_SC // _NW
_LPAD = _L + 16

_sc_mesh = plsc.VectorSubcoreMesh(core_axis_name="c", subcore_axis_name="s")


@functools.partial(
    pl.kernel,
    mesh=_sc_mesh,
    compiler_params=pltpu.CompilerParams(
        use_tc_tiling_on_sc=False, needs_layout_passes=False
    ),
    out_type=jax.ShapeDtypeStruct((_NC, _LANES), jnp.float32),
    scratch_types=[
        pltpu.VMEM((_RPW, _LPAD), jnp.int32),
        pltpu.VMEM((_LANES,), jnp.float32),
        pltpu.VMEM((_NS, _LANES), jnp.float32),
        pltpu.VMEM_SHARED((_NS, _LANES), jnp.float32),
    ],
)
def _bpmll_sc(pk_hbm, out_hbm, pk_v, out_v, all_v, shared):
    cid = lax.axis_index("c")
    sid = lax.axis_index("s")
    wid = sid * _NC + cid
    row0 = wid * _RPW
    pltpu.sync_copy(
        pk_hbm.at[pl.ds(row0, _RPW)], pk_v.at[pl.ds(0, _RPW), pl.ds(0, _L)]
    )
    lanes = lax.iota(jnp.int32, _LANES)
    rows = jnp.minimum(lanes, _RPW - 1)

    def col_body(i, carry):
        sa_neg, sa_pos, cnta, sb_neg, sb_pos, cntb = carry
        c = i * 2
        colsa = jnp.full((_LANES,), 0, jnp.int32) + c
        ya = plsc.load_gather(pk_v, [rows, colsa])
        yb = plsc.load_gather(pk_v, [rows, colsa + 1])
        ta = ya & 1
        tb = yb & 1
        xa = plsc.bitcast(ya & -2, jnp.float32)
        xb = plsc.bitcast(yb & -2, jnp.float32)
        posa = ta == 1
        posb = tb == 1
        ea = jnp.exp(jnp.where(posa, -xa, xa))
        eb = jnp.exp(jnp.where(posb, -xb, xb))
        sa_neg = sa_neg + jnp.where(posa, 0.0, ea)
        sb_neg = sb_neg + jnp.where(posb, 0.0, eb)
        sa_pos = sa_pos + jnp.where(posa, ea, 0.0)
        sb_pos = sb_pos + jnp.where(posb, eb, 0.0)
        cnta = cnta + ta
        cntb = cntb + tb
        return sa_neg, sa_pos, cnta, sb_neg, sb_pos, cntb

    zf = jnp.zeros((_LANES,), jnp.float32)
    zi = jnp.zeros((_LANES,), jnp.int32)
    sa_neg, sa_pos, cnta, sb_neg, sb_pos, cntb = lax.fori_loop(
        0, _L // 2, col_body, (zf, zf, zi, zf, zf, zi)
    )
    s_neg = sa_neg + sb_neg
    s_pos = sa_pos + sb_pos
    cnt = cnta + cntb
    npos = cnt.astype(jnp.float32)
    nneg = jnp.float32(_L) - npos
    partial = jnp.where(
        lanes < _RPW, s_neg * s_pos / (npos * nneg * jnp.float32(_B)), 0.0
    )
    out_v[:] = partial
    pltpu.sync_copy(out_v, shared.at[sid])
    plsc.subcore_barrier()

    @pl.when(sid == 0)
    def _():
        pltpu.sync_copy(shared, all_v)
        acc = jnp.zeros((_LANES,), jnp.float32)
        for i in range(_NS):
            acc = acc + all_v[i, :]
        total = jnp.sum(acc)
        out_v[:] = jnp.zeros((_LANES,), jnp.float32) + total
        pltpu.sync_copy(out_v, out_hbm.at[cid])


_TC_BLK = 128
_TC_STEPS = (_B - _B_SC) // _TC_BLK
_TC_OFF = _B_SC // _TC_BLK


def _tc_body(x_ref, t_ref, o_ref):
    i = pl.program_id(0)

    @pl.when(i == 0)
    def _():
        o_ref[...] = jnp.zeros((1, 1), jnp.float32)

    x = x_ref[...]
    pos = t_ref[...] == 1
    e = jnp.exp(jnp.where(pos, -x, x))
    s_neg = jnp.sum(jnp.where(pos, 0.0, e), axis=1)
    s_pos = jnp.sum(jnp.where(pos, e, 0.0), axis=1)
    npos = jnp.sum(pos.astype(jnp.float32), axis=1)
    loss = s_neg * s_pos / (npos * (jnp.float32(_L) - npos) * jnp.float32(_B))
    o_ref[...] += jnp.sum(loss)[None, None]


_tc_half = pl.pallas_call(
    _tc_body,
    grid=(_TC_STEPS,),
    in_specs=[
        pl.BlockSpec((_TC_BLK, _L), lambda i: (i + _TC_OFF, 0)),
        pl.BlockSpec((_TC_BLK, _L), lambda i: (i + _TC_OFF, 0)),
    ],
    out_specs=pl.BlockSpec((1, 1), lambda i: (0, 0)),
    out_shape=jax.ShapeDtypeStruct((1, 1), jnp.float32),
)


def kernel(input, target):
    tgt32 = target.astype(jnp.int32)
    packed = (
        jax.lax.bitcast_convert_type(input[:_B_SC], jnp.int32) & -2
    ) | tgt32[:_B_SC]
    sc_out = _bpmll_sc(packed)
    tc_out = _tc_half(input, tgt32)
    return sc_out[0, 0] + sc_out[1, 0] + tc_out[0, 0]
